# baseline (speedup 1.0000x reference)
import functools

import jax
import jax.numpy as jnp
from jax import lax
from jax.experimental import pallas as pl
from jax.experimental.pallas import tpu as pltpu
from jax.experimental.pallas import tpu_sc as plsc

_EPS = 1e-5
_NBR = 12
_F = 64


def _sig(x):
    return 0.5 * jnp.tanh(0.5 * x) + 0.5


def _sl(x, n):
    return x[:, n * _F:(n + 1) * _F]


def _phase1_body(node_ref, edge_ref, wtop_ref, wbot_ref, bnu_ref, g1_ref,
                 be1_ref, g2_ref, be2_ref, out_ref):
    node = node_ref[...]
    wbot = wbot_ref[...]
    ba = node.shape[0]
    rows1 = float(ba * _NBR)

    npart = jnp.dot(node, wtop_ref[...], preferred_element_type=jnp.float32)
    npart = npart + bnu_ref[...]

    c1 = [npart + jnp.dot(_sl(edge_ref[...], n), wbot,
                          preferred_element_type=jnp.float32)
          for n in range(_NBR)]

    tot = c1[0]
    for n in range(1, _NBR):
        tot = tot + c1[n]
    mu1 = jnp.sum(tot, axis=0, keepdims=True) / rows1
    v = jnp.sum((c1[0] - mu1) ** 2, axis=0, keepdims=True)
    for n in range(1, _NBR):
        v = v + jnp.sum((c1[n] - mu1) ** 2, axis=0, keepdims=True)
    var1 = v / rows1
    scale1 = g1_ref[...] * lax.rsqrt(var1 + _EPS)
    shift1 = be1_ref[...] - mu1 * scale1

    acc = jnp.zeros((ba, _F), dtype=jnp.float32)
    for n in range(_NBR):
        y = c1[n] * scale1 + shift1
        acc = acc + _sig(y[:, :_F]) * jnp.tanh(y[:, _F:])

    mu2 = jnp.sum(acc, axis=0, keepdims=True) / float(ba)
    var2 = jnp.sum((acc - mu2) ** 2, axis=0, keepdims=True) / float(ba)
    ns = (acc - mu2) * (g2_ref[...] * lax.rsqrt(var2 + _EPS)) + be2_ref[...]
    out_ref[...] = jnp.tanh(node + ns)


def _slj(x, n):
    return x[:, n * 2 * _F:n * 2 * _F + _F]


def _tile12(x):
    return jnp.concatenate([x] * _NBR, axis=1)


def _phase2_body(un_ref, nj_ref, edge_ref, w2_ref, b2_ref, g2b_ref, be2b_ref,
                 out_ref):
    un = un_ref[...]
    nj = nj_ref[...]
    edge = edge_ref[...]
    ba = un.shape[0]
    rows2 = float(ba * _NBR)
    w2 = w2_ref[...]
    b2v = b2_ref[...]

    snj = _slj(nj, 0)
    for n in range(1, _NBR):
        snj = snj + _slj(nj, n)
    mu = jnp.sum(jnp.dot(un * snj, w2, preferred_element_type=jnp.float32),
                 axis=0, keepdims=True) / rows2 + b2v
    c2 = []
    v = jnp.zeros((1, 2 * _F), jnp.float32)
    for n in range(_NBR):
        c2n = jnp.dot(un * _slj(nj, n), w2,
                      preferred_element_type=jnp.float32) + b2v
        c2.append(c2n)
        v = v + jnp.sum((c2n - mu) ** 2, axis=0, keepdims=True)
    scale2 = g2b_ref[...] * lax.rsqrt(v / rows2 + _EPS)
    shift2 = be2b_ref[...] - mu * scale2

    for n in range(_NBR):
        y = c2[n] * scale2 + shift2
        out_ref[:, n * _F:(n + 1) * _F] = (
            _sl(edge, n) + _sig(y[:, :_F]) * jnp.tanh(y[:, _F:]))


def _phase3a_body(un_ref, nj_ref, edge_ref,
                  w3i_ref, w3jk_ref, w3ijkj_ref, b3_ref, g3b_ref, be3b_ref,
                  eb_ref, eq_ref, tb_ref, tq_ref):
    un = un_ref[...]
    nj = nj_ref[...]
    edge = edge_ref[...]
    ba = un.shape[0]
    rows3 = float(ba * _NBR * (_NBR - 1))

    ipart = jnp.dot(un, w3i_ref[...], preferred_element_type=jnp.float32)
    ipart = ipart + b3_ref[...]
    w3jk = w3jk_ref[...]
    w3ie = w3ijkj_ref[...]
    bg, bel_, qg, qe = [], [], [], []
    for n in range(_NBR):
        a = jnp.dot(_slj(nj, n), w3jk, preferred_element_type=jnp.float32)
        b = jnp.dot(_sl(edge, n), w3ie, preferred_element_type=jnp.float32)
        bn = ipart + a[:, :2 * _F] + b[:, :2 * _F]
        bg.append(bn[:, :_F])
        bel_.append(bn[:, _F:])
        qg.append(a[:, 2 * _F:3 * _F] + b[:, 2 * _F:3 * _F])
        qe.append(a[:, 3 * _F:] + b[:, 3 * _F:])

    def _stats(bl, ql, g3b, be3b):
        sb = bl[0]
        sq = ql[0]
        for n in range(1, _NBR):
            sb = sb + bl[n]
            sq = sq + ql[n]
        mu3 = (jnp.sum(sb, axis=0, keepdims=True)
               + jnp.sum(sq, axis=0, keepdims=True)) / (ba * _NBR)
        ss = 2.0 * jnp.sum((sb - float(_NBR) * mu3) * sq, axis=0,
                           keepdims=True)
        for n in range(_NBR):
            bbn = bl[n] - mu3
            qn = ql[n]
            ss = ss + float(_NBR - 1) * jnp.sum(bbn * bbn, axis=0,
                                                keepdims=True)
            ss = ss + float(_NBR - 1) * jnp.sum(qn * qn, axis=0,
                                                keepdims=True)
            ss = ss - 2.0 * jnp.sum(bbn * qn, axis=0, keepdims=True)
        scale3 = g3b * lax.rsqrt(ss / rows3 + _EPS)
        return mu3, scale3, be3b

    mug, sg3, shg = _stats(bg, qg, g3b_ref[:, :_F], be3b_ref[:, :_F])
    mue, se3, she = _stats(bel_, qe, g3b_ref[:, _F:], be3b_ref[:, _F:])

    for n in range(_NBR):
        s = slice(n * _F, (n + 1) * _F)
        bgl = (bg[n] - mug) * sg3 + shg
        eb_ref[:, s] = jnp.exp(-jnp.clip(bgl, -35.0, 35.0))
        eq_ref[:, s] = jnp.exp(-jnp.clip(qg[n] * sg3, -35.0, 35.0))
        tb_ref[:, s] = jnp.tanh((bel_[n] - mue) * se3 + she)
        tq_ref[:, s] = jnp.tanh(qe[n] * se3)


def _phase3b_body(eb_ref, eq_ref, tb_ref, tq_ref, acc_ref):
    eb = [_sl(eb_ref[...], n) for n in range(_NBR)]
    eq = [_sl(eq_ref[...], n) for n in range(_NBR)]
    tb = [_sl(tb_ref[...], n) for n in range(_NBR)]
    tq = [_sl(tq_ref[...], n) for n in range(_NBR)]

    def _pair(n, j):
        num = tb[n] + tq[j]
        den = (1.0 + eb[n] * eq[j]) * (1.0 + tb[n] * tq[j])
        return num / den

    for n in range(_NBR):
        acc = -_pair(n, n)
        for j in range(_NBR):
            acc = acc + _pair(n, j)
        acc_ref[:, n * _F:(n + 1) * _F] = acc


def _phase3c_body(eo_ref, acc_ref, gs_ref, bes_ref, out_ref):
    eo = eo_ref[...]
    accl = acc_ref[...]
    ba = eo.shape[0]
    rows2 = float(ba * _NBR)
    acc = [_sl(accl, n) for n in range(_NBR)]
    t = acc[0]
    for n in range(1, _NBR):
        t = t + acc[n]
    mus = jnp.sum(t, axis=0, keepdims=True) / rows2
    v = jnp.sum((acc[0] - mus) ** 2, axis=0, keepdims=True)
    for n in range(1, _NBR):
        v = v + jnp.sum((acc[n] - mus) ** 2, axis=0, keepdims=True)
    scs = gs_ref[...] * lax.rsqrt(v / rows2 + _EPS)
    shs = bes_ref[...] - mus * scs

    for n in range(_NBR):
        out_ref[:, n * _F:(n + 1) * _F] = jnp.tanh(_sl(eo, n) + acc[n] * scs
                                                   + shs)


def _sc_gather(table, idx):
    info = plsc.get_sparse_core_info()
    nw = info.num_cores * info.num_subcores
    b = idx.shape[0]
    d = table.shape[1]
    bpw = b // nw
    mesh = plsc.VectorSubcoreMesh(core_axis_name="c", subcore_axis_name="s")

    @functools.partial(
        pl.kernel,
        mesh=mesh,
        out_type=jax.ShapeDtypeStruct((b, d), jnp.float32),
        scratch_types=[
            pltpu.VMEM((bpw,), jnp.int32),
            pltpu.VMEM((bpw, d), jnp.float32),
            pltpu.SemaphoreType.DMA,
        ],
    )
    def gk(table_hbm, idx_hbm, out_hbm, idx_v, rows_v, sem):
        wid = lax.axis_index("s") * info.num_cores + lax.axis_index("c")
        base = wid * bpw
        pltpu.sync_copy(idx_hbm.at[pl.ds(base, bpw)], idx_v)
        pltpu.async_copy(table_hbm.at[idx_v], rows_v, sem).wait()
        pltpu.sync_copy(rows_v, out_hbm.at[pl.ds(base, bpw)])

    return gk(table, idx)


def kernel(node_embedding, edge_embedding, nbr_idx, nbr_mask, W_nu, b_nu, g1,
           be1, g2, be2, W2, b2, g2b, be2b, W3, b3, g3b, be3b, gs, bes):
    B, At, Nbr, F = edge_embedding.shape
    BA = B * At
    node2 = node_embedding.reshape(BA, F)
    edge_lane = edge_embedding.reshape(BA, Nbr * F)
    r1 = lambda v: v.reshape(1, -1)

    un2 = pl.pallas_call(
        _phase1_body,
        out_shape=jax.ShapeDtypeStruct((BA, F), jnp.float32),
    )(node2, edge_lane, W_nu[:F], W_nu[F:], r1(b_nu), r1(g1), r1(be1),
      r1(g2), r1(be2))

    offs = (jnp.arange(B, dtype=jnp.int32) * At)[:, None]
    idx_flat = (nbr_idx.reshape(B, At * Nbr) + offs).reshape(B * At * Nbr)
    table = jnp.pad(un2, ((0, 0), (0, F)))
    nj_lane = _sc_gather(table, idx_flat).reshape(BA, Nbr * 2 * F)

    eo_lane = pl.pallas_call(
        _phase2_body,
        out_shape=jax.ShapeDtypeStruct((BA, Nbr * F), jnp.float32),
    )(un2, nj_lane, edge_lane, W2, r1(b2), r1(g2b), r1(be2b))

    w3jk = jnp.concatenate([W3[F:2 * F], W3[2 * F:3 * F]], axis=1)
    w3ijkj = jnp.concatenate([W3[3 * F:4 * F], W3[4 * F:5 * F]], axis=1)
    lane_sh = jax.ShapeDtypeStruct((BA, Nbr * F), jnp.float32)
    eb, eq, tb, tq = pl.pallas_call(
        _phase3a_body,
        out_shape=(lane_sh,) * 4,
    )(un2, nj_lane, edge_lane, W3[:F], w3jk, w3ijkj, r1(b3), r1(g3b),
      r1(be3b))

    rb = 128
    blk = pl.BlockSpec((rb, Nbr * F), lambda i: (i, 0))
    acc = pl.pallas_call(
        _phase3b_body,
        out_shape=lane_sh,
        grid=(BA // rb,),
        in_specs=[blk] * 4,
        out_specs=blk,
    )(eb, eq, tb, tq)

    ue_lane = pl.pallas_call(
        _phase3c_body,
        out_shape=lane_sh,
    )(eo_lane, acc, r1(gs), r1(bes))

    return un2.reshape(B, At, F), ue_lane.reshape(B, At, Nbr, F)


# speedup vs baseline: 1.0919x; 1.0919x over previous
import functools

import jax
import jax.numpy as jnp
from jax import lax
from jax.experimental import pallas as pl
from jax.experimental.pallas import tpu as pltpu
from jax.experimental.pallas import tpu_sc as plsc

_EPS = 1e-5
_NBR = 12
_F = 64


def _sig(x):
    return 0.5 * jnp.tanh(0.5 * x) + 0.5


def _sl(x, n):
    return x[:, n * _F:(n + 1) * _F]


def _phase1_body(node_ref, edge_ref, wtop_ref, wbot_ref, bnu_ref, g1_ref,
                 be1_ref, g2_ref, be2_ref, out_ref):
    node = node_ref[...]
    wbot = wbot_ref[...]
    ba = node.shape[0]
    rows1 = float(ba * _NBR)

    npart = jnp.dot(node, wtop_ref[...], preferred_element_type=jnp.float32)
    npart = npart + bnu_ref[...]

    c1 = [npart + jnp.dot(_sl(edge_ref[...], n), wbot,
                          preferred_element_type=jnp.float32)
          for n in range(_NBR)]

    tot = c1[0]
    for n in range(1, _NBR):
        tot = tot + c1[n]
    mu1 = jnp.sum(tot, axis=0, keepdims=True) / rows1
    v = jnp.sum((c1[0] - mu1) ** 2, axis=0, keepdims=True)
    for n in range(1, _NBR):
        v = v + jnp.sum((c1[n] - mu1) ** 2, axis=0, keepdims=True)
    var1 = v / rows1
    scale1 = g1_ref[...] * lax.rsqrt(var1 + _EPS)
    shift1 = be1_ref[...] - mu1 * scale1

    acc = jnp.zeros((ba, _F), dtype=jnp.float32)
    for n in range(_NBR):
        y = c1[n] * scale1 + shift1
        acc = acc + _sig(y[:, :_F]) * jnp.tanh(y[:, _F:])

    mu2 = jnp.sum(acc, axis=0, keepdims=True) / float(ba)
    var2 = jnp.sum((acc - mu2) ** 2, axis=0, keepdims=True) / float(ba)
    ns = (acc - mu2) * (g2_ref[...] * lax.rsqrt(var2 + _EPS)) + be2_ref[...]
    out_ref[...] = jnp.tanh(node + ns)


def _slj(x, n):
    return x[:, n * 2 * _F:n * 2 * _F + _F]


def _tile12(x):
    return jnp.concatenate([x] * _NBR, axis=1)


def _phase2_body(un_ref, nj_ref, edge_ref, w2_ref, b2_ref, g2b_ref, be2b_ref,
                 out_ref):
    un = un_ref[...]
    nj = nj_ref[...]
    edge = edge_ref[...]
    ba = un.shape[0]
    rows2 = float(ba * _NBR)
    w2 = w2_ref[...]
    b2v = b2_ref[...]

    snj = _slj(nj, 0)
    for n in range(1, _NBR):
        snj = snj + _slj(nj, n)
    mu = jnp.sum(jnp.dot(un * snj, w2, preferred_element_type=jnp.float32),
                 axis=0, keepdims=True) / rows2 + b2v
    c2 = []
    v = jnp.zeros((1, 2 * _F), jnp.float32)
    for n in range(_NBR):
        c2n = jnp.dot(un * _slj(nj, n), w2,
                      preferred_element_type=jnp.float32) + b2v
        c2.append(c2n)
        v = v + jnp.sum((c2n - mu) ** 2, axis=0, keepdims=True)
    scale2 = g2b_ref[...] * lax.rsqrt(v / rows2 + _EPS)
    shift2 = be2b_ref[...] - mu * scale2

    for n in range(_NBR):
        y = c2[n] * scale2 + shift2
        out_ref[:, n * _F:(n + 1) * _F] = (
            _sl(edge, n) + _sig(y[:, :_F]) * jnp.tanh(y[:, _F:]))


def _phase3_body(un_ref, nj_ref, edge_ref, eo_ref,
                 w3i_ref, w3jk_ref, w3ijkj_ref, b3_ref, g3b_ref, be3b_ref,
                 gs_ref, bes_ref, out_ref,
                 eb_ref, eq_ref, tb_ref, tq_ref, acc_ref):
    un = un_ref[...]
    nj = nj_ref[...]
    edge = edge_ref[...]
    ba = un.shape[0]
    rows2 = float(ba * _NBR)
    rows3 = float(ba * _NBR * (_NBR - 1))

    ipart = jnp.dot(un, w3i_ref[...], preferred_element_type=jnp.float32)
    ipart = ipart + b3_ref[...]
    w3jk = w3jk_ref[...]
    w3ie = w3ijkj_ref[...]
    bg, bel_, qg, qe = [], [], [], []
    for n in range(_NBR):
        a = jnp.dot(_slj(nj, n), w3jk, preferred_element_type=jnp.float32)
        b = jnp.dot(_sl(edge, n), w3ie, preferred_element_type=jnp.float32)
        bn = ipart + a[:, :2 * _F] + b[:, :2 * _F]
        bg.append(bn[:, :_F])
        bel_.append(bn[:, _F:])
        qg.append(a[:, 2 * _F:3 * _F] + b[:, 2 * _F:3 * _F])
        qe.append(a[:, 3 * _F:] + b[:, 3 * _F:])

    def _stats(bl, ql, g3b, be3b):
        sb = bl[0]
        sq = ql[0]
        for n in range(1, _NBR):
            sb = sb + bl[n]
            sq = sq + ql[n]
        mu3 = (jnp.sum(sb, axis=0, keepdims=True)
               + jnp.sum(sq, axis=0, keepdims=True)) / (ba * _NBR)
        ss = 2.0 * jnp.sum((sb - float(_NBR) * mu3) * sq, axis=0,
                           keepdims=True)
        for n in range(_NBR):
            bbn = bl[n] - mu3
            qn = ql[n]
            ss = ss + float(_NBR - 1) * jnp.sum(bbn * bbn, axis=0,
                                                keepdims=True)
            ss = ss + float(_NBR - 1) * jnp.sum(qn * qn, axis=0,
                                                keepdims=True)
            ss = ss - 2.0 * jnp.sum(bbn * qn, axis=0, keepdims=True)
        scale3 = g3b * lax.rsqrt(ss / rows3 + _EPS)
        return mu3, scale3, be3b

    mug, sg3, shg = _stats(bg, qg, g3b_ref[:, :_F], be3b_ref[:, :_F])
    mue, se3, she = _stats(bel_, qe, g3b_ref[:, _F:], be3b_ref[:, _F:])

    for n in range(_NBR):
        s = slice(n * _F, (n + 1) * _F)
        bgl = (bg[n] - mug) * sg3 + shg
        eb_ref[:, s] = jnp.exp(-jnp.clip(bgl, -35.0, 35.0))
        eq_ref[:, s] = jnp.exp(-jnp.clip(qg[n] * sg3, -35.0, 35.0))
        tb_ref[:, s] = jnp.tanh((bel_[n] - mue) * se3 + she)
        tq_ref[:, s] = jnp.tanh(qe[n] * se3)

    rb = 128

    def _blk(i, carry):
        r = pl.ds(i * rb, rb)
        ebb = eb_ref[r, :]
        eqb = eq_ref[r, :]
        tbb = tb_ref[r, :]
        tqb = tq_ref[r, :]

        def _pair(n, j):
            num = _sl(tbb, n) + _sl(tqb, j)
            den = ((1.0 + _sl(ebb, n) * _sl(eqb, j))
                   * (1.0 + _sl(tbb, n) * _sl(tqb, j)))
            return num / den

        for n in range(_NBR):
            a = -_pair(n, n)
            for j in range(_NBR):
                a = a + _pair(n, j)
            acc_ref[r, n * _F:(n + 1) * _F] = a
        return carry

    lax.fori_loop(0, ba // rb, _blk, 0, unroll=False)

    eo = eo_ref[...]
    accl = acc_ref[...]
    acc = [_sl(accl, n) for n in range(_NBR)]
    t = acc[0]
    for n in range(1, _NBR):
        t = t + acc[n]
    mus = jnp.sum(t, axis=0, keepdims=True) / rows2
    v = jnp.sum((acc[0] - mus) ** 2, axis=0, keepdims=True)
    for n in range(1, _NBR):
        v = v + jnp.sum((acc[n] - mus) ** 2, axis=0, keepdims=True)
    scs = gs_ref[...] * lax.rsqrt(v / rows2 + _EPS)
    shs = bes_ref[...] - mus * scs

    for n in range(_NBR):
        out_ref[:, n * _F:(n + 1) * _F] = jnp.tanh(_sl(eo, n) + acc[n] * scs
                                                   + shs)


def _sc_gather(table, idx):
    info = plsc.get_sparse_core_info()
    nw = info.num_cores * info.num_subcores
    b = idx.shape[0]
    d = table.shape[1]
    bpw = b // nw
    mesh = plsc.VectorSubcoreMesh(core_axis_name="c", subcore_axis_name="s")

    @functools.partial(
        pl.kernel,
        mesh=mesh,
        out_type=jax.ShapeDtypeStruct((b, d), jnp.float32),
        scratch_types=[
            pltpu.VMEM((bpw,), jnp.int32),
            pltpu.VMEM((bpw, d), jnp.float32),
            pltpu.SemaphoreType.DMA,
        ],
    )
    def gk(table_hbm, idx_hbm, out_hbm, idx_v, rows_v, sem):
        wid = lax.axis_index("s") * info.num_cores + lax.axis_index("c")
        base = wid * bpw
        pltpu.sync_copy(idx_hbm.at[pl.ds(base, bpw)], idx_v)
        pltpu.async_copy(table_hbm.at[idx_v], rows_v, sem).wait()
        pltpu.sync_copy(rows_v, out_hbm.at[pl.ds(base, bpw)])

    return gk(table, idx)


def kernel(node_embedding, edge_embedding, nbr_idx, nbr_mask, W_nu, b_nu, g1,
           be1, g2, be2, W2, b2, g2b, be2b, W3, b3, g3b, be3b, gs, bes):
    B, At, Nbr, F = edge_embedding.shape
    BA = B * At
    node2 = node_embedding.reshape(BA, F)
    edge_lane = edge_embedding.reshape(BA, Nbr * F)
    r1 = lambda v: v.reshape(1, -1)

    un2 = pl.pallas_call(
        _phase1_body,
        out_shape=jax.ShapeDtypeStruct((BA, F), jnp.float32),
    )(node2, edge_lane, W_nu[:F], W_nu[F:], r1(b_nu), r1(g1), r1(be1),
      r1(g2), r1(be2))

    offs = (jnp.arange(B, dtype=jnp.int32) * At)[:, None]
    idx_flat = (nbr_idx.reshape(B, At * Nbr) + offs).reshape(B * At * Nbr)
    table = jnp.pad(un2, ((0, 0), (0, F)))
    nj_lane = _sc_gather(table, idx_flat).reshape(BA, Nbr * 2 * F)

    eo_lane = pl.pallas_call(
        _phase2_body,
        out_shape=jax.ShapeDtypeStruct((BA, Nbr * F), jnp.float32),
    )(un2, nj_lane, edge_lane, W2, r1(b2), r1(g2b), r1(be2b))

    w3jk = jnp.concatenate([W3[F:2 * F], W3[2 * F:3 * F]], axis=1)
    w3ijkj = jnp.concatenate([W3[3 * F:4 * F], W3[4 * F:5 * F]], axis=1)
    lane_sh = jax.ShapeDtypeStruct((BA, Nbr * F), jnp.float32)
    scratch = pltpu.VMEM((BA, Nbr * F), jnp.float32)
    ue_lane = pl.pallas_call(
        _phase3_body,
        out_shape=lane_sh,
        scratch_shapes=[scratch] * 5,
    )(un2, nj_lane, edge_lane, eo_lane, W3[:F], w3jk, w3ijkj, r1(b3),
      r1(g3b), r1(be3b), r1(gs), r1(bes))

    return un2.reshape(B, At, F), ue_lane.reshape(B, At, Nbr, F)
